# baseline (speedup 1.0000x reference)
import functools

import jax
import jax.numpy as jnp
from jax import lax
from jax.experimental import pallas as pl
from jax.experimental.pallas import tpu as pltpu
from jax.experimental.pallas import tpu_sc as plsc

_NUM_CORES = 2
_NUM_SUBCORES = 16
_NW = _NUM_CORES * _NUM_SUBCORES
_D = 128
_SEQ = 50
_SEQ_PAD = 56
_BPI = 2
_CHUNK = _SEQ_PAD * _BPI
_NBUF = 4
_DEPTH = 3


@functools.lru_cache(maxsize=None)
def _make_lookup(n_batch: int):
    assert n_batch % (_NW * _BPI) == 0
    b_per_w = n_batch // _NW
    nk = b_per_w // _BPI
    idx_per_w = b_per_w * _SEQ_PAD

    mesh = plsc.VectorSubcoreMesh(core_axis_name="c", subcore_axis_name="s")

    @functools.partial(
        pl.kernel,
        mesh=mesh,
        out_type=jax.ShapeDtypeStruct((n_batch, _SEQ, _D), jnp.float32),
        scratch_types=[
            pltpu.VMEM((idx_per_w,), jnp.int32),
            pltpu.VMEM((_NBUF, _CHUNK, _D), jnp.float32),
            pltpu.SemaphoreType.DMA((_NBUF,)),
            pltpu.SemaphoreType.DMA,
        ],
        compiler_params=pltpu.CompilerParams(use_tc_tiling_on_sc=True),
    )
    def lookup(idx_hbm, table_hbm, out_hbm, idx_v, rows_v, gsem, ssem):
        wid = lax.axis_index("s") * _NUM_CORES + lax.axis_index("c")
        batch0 = wid * b_per_w
        pltpu.sync_copy(idx_hbm.at[pl.ds(batch0 * _SEQ_PAD, idx_per_w)], idx_v)

        def gather(k, slot):
            pltpu.async_copy(
                table_hbm.at[idx_v.at[pl.ds(k * _CHUNK, _CHUNK)]],
                rows_v.at[slot],
                gsem.at[slot],
            )

        for j in range(_DEPTH):
            gather(j, j)

        def step(k, carry):
            slot = k % _NBUF

            @pl.when(k >= 1)
            def _drain_prev():
                for _ in range(_BPI):
                    pltpu.make_async_copy(
                        rows_v.at[0, pl.ds(0, _SEQ)], out_hbm.at[0], ssem
                    ).wait()

            nxt = k + _DEPTH

            @pl.when(nxt < nk)
            def _issue_next():
                gather(nxt, nxt % _NBUF)

            pltpu.make_async_copy(
                table_hbm.at[idx_v.at[pl.ds(k * _CHUNK, _CHUNK)]],
                rows_v.at[slot],
                gsem.at[slot],
            ).wait()

            for i in range(_BPI):
                pltpu.async_copy(
                    rows_v.at[slot, pl.ds(i * _SEQ_PAD, _SEQ)],
                    out_hbm.at[batch0 + k * _BPI + i],
                    ssem,
                )
            return carry

        lax.fori_loop(0, nk, step, 0)

        for _ in range(_BPI):
            pltpu.make_async_copy(
                rows_v.at[0, pl.ds(0, _SEQ)], out_hbm.at[0], ssem
            ).wait()

    return lookup


def kernel(token_ids, embed_mat):
    b, s = token_ids.shape
    assert s == _SEQ
    idx_flat = jnp.pad(
        token_ids.astype(jnp.int32), ((0, 0), (0, _SEQ_PAD - s))
    ).reshape(-1)
    return _make_lookup(b)(idx_flat, embed_mat)


# speedup vs baseline: 7.3289x; 7.3289x over previous
import functools

import jax
import jax.numpy as jnp
from jax import lax
from jax.experimental import pallas as pl
from jax.experimental.pallas import tpu as pltpu
from jax.experimental.pallas import tpu_sc as plsc

_NUM_CORES = 2
_NUM_SUBCORES = 16
_NW = _NUM_CORES * _NUM_SUBCORES
_D = 128
_SEQ = 50
_SEQ_PAD = 56
_BPI = 2
_CHUNK = _SEQ_PAD * _BPI
_NBUF = 4
_DEPTH = 3


@functools.lru_cache(maxsize=None)
def _make_lookup(n_batch: int):
    assert n_batch % (_NW * _BPI) == 0
    b_per_w = n_batch // _NW
    nk = b_per_w // _BPI
    idx_per_w = b_per_w * _SEQ_PAD

    mesh = plsc.VectorSubcoreMesh(core_axis_name="c", subcore_axis_name="s")

    @functools.partial(
        pl.kernel,
        mesh=mesh,
        out_type=jax.ShapeDtypeStruct((n_batch, _SEQ, _D), jnp.float32),
        scratch_types=[
            pltpu.VMEM((idx_per_w,), jnp.int32),
            pltpu.VMEM((_NBUF, _CHUNK, _D), jnp.float32),
            pltpu.SemaphoreType.DMA((_NBUF,)),
            pltpu.SemaphoreType.DMA,
        ],
        compiler_params=pltpu.CompilerParams(use_tc_tiling_on_sc=True),
    )
    def lookup(idx_hbm, table_hbm, out_hbm, idx_v, rows_v, gsem, ssem):
        wid = lax.axis_index("s") * _NUM_CORES + lax.axis_index("c")
        batch0 = wid * b_per_w
        pltpu.sync_copy(idx_hbm.at[pl.ds(batch0 * _SEQ_PAD, idx_per_w)], idx_v)

        def gather(k, slot):
            pltpu.async_copy(
                table_hbm.at[idx_v.at[pl.ds(k * _CHUNK, _CHUNK)]],
                rows_v.at[slot],
                gsem.at[slot],
            )

        for j in range(_DEPTH):
            gather(j, j)

        def step(k, carry):
            slot = k % _NBUF

            @pl.when(k >= 1)
            def _drain_prev():
                for _ in range(_BPI):
                    pltpu.make_async_copy(
                        rows_v.at[0, pl.ds(0, _SEQ)], out_hbm.at[0], ssem
                    ).wait()

            nxt = k + _DEPTH

            @pl.when(nxt < nk)
            def _issue_next():
                gather(nxt, nxt % _NBUF)

            pltpu.make_async_copy(
                table_hbm.at[idx_v.at[pl.ds(k * _CHUNK, _CHUNK)]],
                rows_v.at[slot],
                gsem.at[slot],
            ).wait()

            for i in range(_BPI):
                pltpu.async_copy(
                    rows_v.at[slot, pl.ds(i * _SEQ_PAD, _SEQ)],
                    out_hbm.at[batch0 + k * _BPI + i],
                    ssem,
                )
            return carry

        lax.fori_loop(0, nk, step, 0)

        for _ in range(_BPI):
            pltpu.make_async_copy(
                rows_v.at[0, pl.ds(0, _SEQ)], out_hbm.at[0], ssem
            ).wait()

    return lookup


def kernel(token_ids, embed_mat):
    b, s = token_ids.shape
    assert s == _SEQ
    idx_flat = jnp.pad(
        token_ids.astype(jnp.int32), ((0, 0), (0, _SEQ_PAD - s)), mode="wrap"
    ).reshape(-1)
    return _make_lookup(b)(idx_flat, embed_mat)


# speedup vs baseline: 13.3136x; 1.8166x over previous
import functools

import jax
import jax.numpy as jnp
from jax import lax
from jax.experimental import pallas as pl
from jax.experimental.pallas import tpu as pltpu
from jax.experimental.pallas import tpu_sc as plsc

_NUM_CORES = 2
_NUM_SUBCORES = 16
_NW = _NUM_CORES * _NUM_SUBCORES
_D = 128
_CHUNK = 128
_NBUF = 4
_DEPTH = 3


@functools.lru_cache(maxsize=None)
def _make_lookup(n_tokens: int):
    assert n_tokens % (_NW * _CHUNK) == 0
    idx_per_w = n_tokens // _NW
    nk = idx_per_w // _CHUNK

    mesh = plsc.VectorSubcoreMesh(core_axis_name="c", subcore_axis_name="s")

    @functools.partial(
        pl.kernel,
        mesh=mesh,
        out_type=jax.ShapeDtypeStruct((n_tokens, _D), jnp.float32),
        scratch_types=[
            pltpu.VMEM((idx_per_w,), jnp.int32),
            pltpu.VMEM((_NBUF, _CHUNK, _D), jnp.float32),
            pltpu.SemaphoreType.DMA((_NBUF,)),
            pltpu.SemaphoreType.DMA,
        ],
    )
    def lookup(idx_hbm, table_hbm, out_hbm, idx_v, rows_v, gsem, ssem):
        wid = lax.axis_index("s") * _NUM_CORES + lax.axis_index("c")
        base = wid * idx_per_w
        pltpu.sync_copy(idx_hbm.at[pl.ds(base, idx_per_w)], idx_v)

        def gather(k, slot):
            pltpu.async_copy(
                table_hbm.at[idx_v.at[pl.ds(k * _CHUNK, _CHUNK)]],
                rows_v.at[slot],
                gsem.at[slot],
            )

        for j in range(_DEPTH):
            gather(j, j)

        def step(k, carry):
            slot = k % _NBUF

            @pl.when(k >= 1)
            def _drain_prev():
                pltpu.make_async_copy(
                    rows_v.at[0], out_hbm.at[pl.ds(0, _CHUNK)], ssem
                ).wait()

            nxt = k + _DEPTH

            @pl.when(nxt < nk)
            def _issue_next():
                gather(nxt, nxt % _NBUF)

            pltpu.make_async_copy(
                table_hbm.at[idx_v.at[pl.ds(k * _CHUNK, _CHUNK)]],
                rows_v.at[slot],
                gsem.at[slot],
            ).wait()
            pltpu.async_copy(
                rows_v.at[slot],
                out_hbm.at[pl.ds(base + k * _CHUNK, _CHUNK)],
                ssem,
            )
            return carry

        lax.fori_loop(0, nk, step, 0)

        pltpu.make_async_copy(
            rows_v.at[0], out_hbm.at[pl.ds(0, _CHUNK)], ssem
        ).wait()

    return lookup


def kernel(token_ids, embed_mat):
    b, s = token_ids.shape
    n = b * s
    idx = token_ids.astype(jnp.int32).T.reshape(-1)
    out = _make_lookup(n)(idx, embed_mat)
    return out.reshape(s, b, _D).transpose(1, 0, 2)


# speedup vs baseline: 13.4091x; 1.0072x over previous
import functools

import jax
import jax.numpy as jnp
from jax import lax
from jax.experimental import pallas as pl
from jax.experimental.pallas import tpu as pltpu
from jax.experimental.pallas import tpu_sc as plsc

_NUM_CORES = 2
_NUM_SUBCORES = 16
_NW = _NUM_CORES * _NUM_SUBCORES
_D = 128
_CHUNK = 128
_NBUF = 6
_DEPTH = 4


@functools.lru_cache(maxsize=None)
def _make_lookup(n_tokens: int):
    assert n_tokens % (_NW * _CHUNK) == 0
    idx_per_w = n_tokens // _NW
    nk = idx_per_w // _CHUNK

    mesh = plsc.VectorSubcoreMesh(core_axis_name="c", subcore_axis_name="s")

    @functools.partial(
        pl.kernel,
        mesh=mesh,
        out_type=jax.ShapeDtypeStruct((n_tokens, _D), jnp.float32),
        scratch_types=[
            pltpu.VMEM((idx_per_w,), jnp.int32),
            pltpu.VMEM((_NBUF, _CHUNK, _D), jnp.float32),
            pltpu.SemaphoreType.DMA((_NBUF,)),
            pltpu.SemaphoreType.DMA((2,)),
        ],
    )
    def lookup(idx_hbm, table_hbm, out_hbm, idx_v, rows_v, gsem, ssem):
        wid = lax.axis_index("s") * _NUM_CORES + lax.axis_index("c")
        base = wid * idx_per_w
        pltpu.sync_copy(idx_hbm.at[pl.ds(base, idx_per_w)], idx_v)

        def gather(k, slot):
            pltpu.async_copy(
                table_hbm.at[idx_v.at[pl.ds(k * _CHUNK, _CHUNK)]],
                rows_v.at[slot],
                gsem.at[slot],
            )

        for j in range(_DEPTH):
            gather(j, j)

        def step(k, carry):
            slot = k % _NBUF

            @pl.when(k >= 2)
            def _drain_prev():
                pltpu.make_async_copy(
                    rows_v.at[0], out_hbm.at[pl.ds(0, _CHUNK)], ssem.at[k % 2]
                ).wait()

            nxt = k + _DEPTH

            @pl.when(nxt < nk)
            def _issue_next():
                gather(nxt, nxt % _NBUF)

            pltpu.make_async_copy(
                table_hbm.at[idx_v.at[pl.ds(k * _CHUNK, _CHUNK)]],
                rows_v.at[slot],
                gsem.at[slot],
            ).wait()
            pltpu.async_copy(
                rows_v.at[slot],
                out_hbm.at[pl.ds(base + k * _CHUNK, _CHUNK)],
                ssem.at[k % 2],
            )
            return carry

        lax.fori_loop(0, nk, step, 0)

        for j in range(2):
            pltpu.make_async_copy(
                rows_v.at[0], out_hbm.at[pl.ds(0, _CHUNK)], ssem.at[j]
            ).wait()

    return lookup


def kernel(token_ids, embed_mat):
    b, s = token_ids.shape
    n = b * s
    idx = token_ids.astype(jnp.int32).T.reshape(-1)
    out = _make_lookup(n)(idx, embed_mat)
    return out.reshape(s, b, _D).transpose(1, 0, 2)


# speedup vs baseline: 13.4259x; 1.0013x over previous
import functools

import jax
import jax.numpy as jnp
from jax import lax
from jax.experimental import pallas as pl
from jax.experimental.pallas import tpu as pltpu
from jax.experimental.pallas import tpu_sc as plsc

_NUM_CORES = 2
_NUM_SUBCORES = 16
_NW = _NUM_CORES * _NUM_SUBCORES
_D = 128
_CHUNK = 128
_NBUF = 7
_DEPTH = 5


@functools.lru_cache(maxsize=None)
def _make_lookup(n_tokens: int):
    assert n_tokens % (_NW * _CHUNK) == 0
    idx_per_w = n_tokens // _NW
    nk = idx_per_w // _CHUNK

    mesh = plsc.VectorSubcoreMesh(core_axis_name="c", subcore_axis_name="s")

    @functools.partial(
        pl.kernel,
        mesh=mesh,
        out_type=jax.ShapeDtypeStruct((n_tokens, _D), jnp.float32),
        scratch_types=[
            pltpu.VMEM((idx_per_w,), jnp.int32),
            pltpu.VMEM((_NBUF, _CHUNK, _D), jnp.float32),
            pltpu.SemaphoreType.DMA((_NBUF,)),
            pltpu.SemaphoreType.DMA((2,)),
        ],
    )
    def lookup(idx_hbm, table_hbm, out_hbm, idx_v, rows_v, gsem, ssem):
        wid = lax.axis_index("s") * _NUM_CORES + lax.axis_index("c")
        base = wid * idx_per_w
        pltpu.sync_copy(idx_hbm.at[pl.ds(base, idx_per_w)], idx_v)

        def gather(k, slot):
            pltpu.async_copy(
                table_hbm.at[idx_v.at[pl.ds(k * _CHUNK, _CHUNK)]],
                rows_v.at[slot],
                gsem.at[slot],
            )

        for j in range(_DEPTH):
            gather(j, j)

        def step(k, carry):
            slot = k % _NBUF

            @pl.when(k >= 2)
            def _drain_prev():
                pltpu.make_async_copy(
                    rows_v.at[0], out_hbm.at[pl.ds(0, _CHUNK)], ssem.at[k % 2]
                ).wait()

            nxt = k + _DEPTH

            @pl.when(nxt < nk)
            def _issue_next():
                gather(nxt, nxt % _NBUF)

            pltpu.make_async_copy(
                table_hbm.at[idx_v.at[pl.ds(k * _CHUNK, _CHUNK)]],
                rows_v.at[slot],
                gsem.at[slot],
            ).wait()
            pltpu.async_copy(
                rows_v.at[slot],
                out_hbm.at[pl.ds(base + k * _CHUNK, _CHUNK)],
                ssem.at[k % 2],
            )
            return carry

        lax.fori_loop(0, nk, step, 0)

        for j in range(2):
            pltpu.make_async_copy(
                rows_v.at[0], out_hbm.at[pl.ds(0, _CHUNK)], ssem.at[j]
            ).wait()

    return lookup


def kernel(token_ids, embed_mat):
    b, s = token_ids.shape
    n = b * s
    idx = token_ids.astype(jnp.int32).T.reshape(-1)
    out = _make_lookup(n)(idx, embed_mat)
    return out.reshape(s, b, _D).transpose(1, 0, 2)
